# Initial kernel scaffold; baseline (speedup 1.0000x reference)
import jax
import jax.numpy as jnp
from jax.experimental import pallas as pl


def kernel(x, edge_index, edge_attr, u, batch, W1, b1, W2, b2):
    raise NotImplementedError("write your pallas kernel here")


# speedup vs baseline: 3.8767x; 3.8767x over previous
import functools

import jax
import jax.numpy as jnp
from jax import lax
from jax.experimental import pallas as pl
from jax.experimental.pallas import tpu as pltpu
from jax.experimental.pallas import tpu_sc as plsc

NC = 2
NS = 16
NW = NC * NS
L = 16
ACCW = 32


def _mm_body(x_ref, w_ref, o_ref):
    o_ref[...] = jnp.dot(x_ref[...], w_ref[...],
                         preferred_element_type=jnp.float32)


def _mm_bias_body(x_ref, w_ref, b_ref, o_ref):
    o_ref[...] = jnp.dot(x_ref[...], w_ref[...],
                         preferred_element_type=jnp.float32) + b_ref[...]


def _sc_scatter_body(NCH, CH, NPS, npad,
                     xw1_hbm, ew_hbm, row_hbm, col_hbm, out_hbm,
                     row_buf, col_buf, ew_buf, gath_buf, val_buf, node_buf,
                     acc_sh, sem):
    c = lax.axis_index("c")
    s = lax.axis_index("s")
    wid = s * NC + c

    zvec = jnp.zeros((L,), jnp.float32)

    def zero_row(i, carry):
        node_buf[i, pl.ds(0, L)] = zvec
        node_buf[i, pl.ds(L, L)] = zvec
        return carry

    lax.fori_loop(0, NPS, zero_row, 0)
    pltpu.sync_copy(node_buf, acc_sh.at[pl.ds(s * NPS, NPS)])

    cvec = jnp.where(lax.iota(jnp.int32, L) == 0, 1.0, 0.0)

    def count_row(i, carry):
        val_buf[i, pl.ds(L, L)] = cvec
        return carry

    lax.fori_loop(0, CH, count_row, 0)

    pltpu.sync_copy(row_hbm.at[wid], row_buf)
    pltpu.sync_copy(col_hbm.at[wid], col_buf)

    plsc.subcore_barrier()

    def chunk(k, carry):
        pltpu.sync_copy(ew_hbm.at[wid, pl.ds(k * CH, CH)], ew_buf)
        pltpu.async_copy(xw1_hbm.at[row_buf.at[k]], gath_buf, sem).wait()

        def edge(i, carry2):
            v = jnp.maximum(gath_buf[i] + ew_buf[i], 0.0)
            val_buf[i, pl.ds(0, L)] = v
            return carry2

        lax.fori_loop(0, CH, edge, 0, unroll=8)
        pltpu.sync_copy(val_buf, acc_sh.at[col_buf.at[k]], add=True)
        return carry

    lax.fori_loop(0, NCH, chunk, 0)

    plsc.subcore_barrier()

    pltpu.sync_copy(acc_sh.at[pl.ds(s * NPS, NPS)], node_buf)
    pltpu.sync_copy(node_buf, out_hbm.at[pl.ds(c * npad + s * NPS, NPS)])


def _node_body(fx, x_ref, p_ref, w_ref, b_ref, o_ref):
    p0 = p_ref[0]
    p1 = p_ref[1]
    acc = p0[:, :L] + p1[:, :L]
    cnt = p0[:, L:L + 1] + p1[:, L:L + 1]
    mean = acc / jnp.maximum(cnt, 1.0)
    h = jnp.dot(x_ref[...], w_ref[:fx], preferred_element_type=jnp.float32)
    h = h + jnp.dot(mean, w_ref[fx:], preferred_element_type=jnp.float32)
    o_ref[...] = jnp.maximum(h + b_ref[...], 0.0)


def kernel(x, edge_index, edge_attr, u, batch, W1, b1, W2, b2):
    n, fx = x.shape
    e, fe = edge_attr.shape
    dout = W1.shape[1]
    assert dout == L and e % NW == 0 and n % NS == 0

    epw = e // NW
    ch = 80
    nch = epw // ch
    assert nch * ch == epw
    npad = -(-n // (NS * 8)) * (NS * 8)
    nps = npad // NS

    row = edge_index[0].reshape(NW, nch, ch)
    col = edge_index[1].reshape(NW, nch, ch)
    W1a = W1[:fx]
    W1b = W1[fx:]

    rb = 1000
    xw1 = pl.pallas_call(
        _mm_body,
        grid=(n // rb,),
        in_specs=[pl.BlockSpec((rb, fx), lambda i: (i, 0)),
                  pl.BlockSpec((fx, dout), lambda i: (0, 0))],
        out_specs=pl.BlockSpec((rb, dout), lambda i: (i, 0)),
        out_shape=jax.ShapeDtypeStruct((n, dout), jnp.float32),
    )(x, W1a)

    eb = 4000
    ew = pl.pallas_call(
        _mm_bias_body,
        grid=(e // eb,),
        in_specs=[pl.BlockSpec((eb, fe), lambda i: (i, 0)),
                  pl.BlockSpec((fe, dout), lambda i: (0, 0)),
                  pl.BlockSpec((1, dout), lambda i: (0, 0))],
        out_specs=pl.BlockSpec((eb, dout), lambda i: (i, 0)),
        out_shape=jax.ShapeDtypeStruct((e, dout), jnp.float32),
    )(edge_attr, W1b, b1.reshape(1, dout))
    ew3 = ew.reshape(NW, epw, dout)

    mesh = plsc.VectorSubcoreMesh(core_axis_name="c", subcore_axis_name="s",
                                  num_cores=NC, num_subcores=NS)
    sc_fn = pl.kernel(
        functools.partial(_sc_scatter_body, nch, ch, nps, npad),
        out_type=jax.ShapeDtypeStruct((NC * npad, ACCW), jnp.float32),
        mesh=mesh,
        compiler_params=pltpu.CompilerParams(use_tc_tiling_on_sc=False),
        scratch_types=[
            pltpu.VMEM((nch, ch), jnp.int32),
            pltpu.VMEM((nch, ch), jnp.int32),
            pltpu.VMEM((ch, dout), jnp.float32),
            pltpu.VMEM((ch, dout), jnp.float32),
            pltpu.VMEM((ch, ACCW), jnp.float32),
            pltpu.VMEM((nps, ACCW), jnp.float32),
            pltpu.VMEM_SHARED((npad, ACCW), jnp.float32),
            pltpu.SemaphoreType.DMA,
        ],
    )
    parts = sc_fn(xw1, ew3, row, col).reshape(NC, npad, ACCW)

    out = pl.pallas_call(
        functools.partial(_node_body, fx),
        grid=(n // rb,),
        in_specs=[pl.BlockSpec((rb, fx), lambda i: (i, 0)),
                  pl.BlockSpec((NC, rb, ACCW), lambda i: (0, i, 0)),
                  pl.BlockSpec((fx + dout, dout), lambda i: (0, 0)),
                  pl.BlockSpec((1, dout), lambda i: (0, 0))],
        out_specs=pl.BlockSpec((rb, dout), lambda i: (i, 0)),
        out_shape=jax.ShapeDtypeStruct((n, dout), jnp.float32),
    )(x, parts, W2, b2.reshape(1, dout))
    return out


# speedup vs baseline: 5.7775x; 1.4903x over previous
import functools

import jax
import jax.numpy as jnp
from jax import lax
from jax.experimental import pallas as pl
from jax.experimental.pallas import tpu as pltpu
from jax.experimental.pallas import tpu_sc as plsc

NC = 2
NS = 16
NW = NC * NS
L = 16
ACCW = 32
CH = 128


def _mm_body(x_ref, w_ref, o_ref):
    o_ref[...] = jnp.dot(x_ref[...], w_ref[...],
                         preferred_element_type=jnp.float32)


def _mm_bias_body(x_ref, w_ref, b_ref, o_ref):
    o_ref[...] = jnp.dot(x_ref[...], w_ref[...],
                         preferred_element_type=jnp.float32) + b_ref[...]


def _sc_scatter_body(NCH, NPS, npad, epw8,
                     xw1_hbm, ew_hbm, row_hbm, col_hbm, out_hbm,
                     row_buf, col_buf, ew_buf, gath_buf, val_buf, node_buf,
                     acc_sh, sem):
    c = lax.axis_index("c")
    s = lax.axis_index("s")
    wid = s * NC + c

    zvec = jnp.zeros((L,), jnp.float32)

    def zero_row(i, carry):
        node_buf[i, pl.ds(0, L)] = zvec
        node_buf[i, pl.ds(L, L)] = zvec
        return carry

    lax.fori_loop(0, NPS, zero_row, 0)
    pltpu.sync_copy(node_buf, acc_sh.at[pl.ds(s * NPS, NPS)])

    cvec = jnp.where(lax.iota(jnp.int32, L) == 0, 1.0, 0.0)

    def count_row(i, carry):
        val_buf[i, pl.ds(L, L)] = cvec
        return carry

    lax.fori_loop(0, CH, count_row, 0)

    pltpu.sync_copy(row_hbm.at[wid], row_buf)
    pltpu.sync_copy(col_hbm.at[wid], col_buf)

    plsc.subcore_barrier()

    def chunk(k, carry):
        pltpu.sync_copy(
            ew_hbm.at[pl.ds(wid * epw8 + k * (CH // 8), CH // 8)], ew_buf)
        pltpu.async_copy(xw1_hbm.at[row_buf.at[k]], gath_buf, sem).wait()

        def pack8(j, carry2):
            for t in range(8):
                i = j * 8 + t
                v = gath_buf[i] + ew_buf[j, pl.ds(t * L, L)]
                val_buf[i, pl.ds(0, L)] = jnp.maximum(v, 0.0)
            return carry2

        lax.fori_loop(0, CH // 8, pack8, 0)
        pltpu.sync_copy(val_buf, acc_sh.at[col_buf.at[k]], add=True)
        return carry

    lax.fori_loop(0, NCH, chunk, 0)

    plsc.subcore_barrier()

    pltpu.sync_copy(acc_sh.at[pl.ds(s * NPS, NPS)], node_buf)
    pltpu.sync_copy(node_buf, out_hbm.at[pl.ds(c * npad + s * NPS, NPS)])


def _node_body(fx, x_ref, p_ref, w_ref, b_ref, o_ref):
    p0 = p_ref[0]
    p1 = p_ref[1]
    acc = p0[:, :L] + p1[:, :L]
    cnt = p0[:, L:L + 1] + p1[:, L:L + 1]
    mean = acc / jnp.maximum(cnt, 1.0)
    h = jnp.dot(x_ref[...], w_ref[:fx], preferred_element_type=jnp.float32)
    h = h + jnp.dot(mean, w_ref[fx:], preferred_element_type=jnp.float32)
    o_ref[...] = jnp.maximum(h + b_ref[...], 0.0)


def kernel(x, edge_index, edge_attr, u, batch, W1, b1, W2, b2):
    n, fx = x.shape
    e, fe = edge_attr.shape
    dout = W1.shape[1]
    assert dout == L and e % NW == 0 and fe == 16

    epw = e // NW
    nch = -(-epw // CH)
    epw_p = nch * CH
    npad = -(-n // (NS * 8)) * (NS * 8)
    nps = npad // NS
    assert npad > n

    ri = jnp.pad(edge_index[0].reshape(NW, epw), ((0, 0), (0, epw_p - epw)))
    row = ri.reshape(NW, nch, CH)
    ci = jnp.pad(edge_index[1].reshape(NW, epw), ((0, 0), (0, epw_p - epw)),
                 constant_values=n)
    col = ci.reshape(NW, nch, CH)

    W1a = W1[:fx]
    W1b = W1[fx:]

    rb = 1000
    xw1 = pl.pallas_call(
        _mm_body,
        grid=(n // rb,),
        in_specs=[pl.BlockSpec((rb, fx), lambda i: (i, 0)),
                  pl.BlockSpec((fx, dout), lambda i: (0, 0))],
        out_specs=pl.BlockSpec((rb, dout), lambda i: (i, 0)),
        out_shape=jax.ShapeDtypeStruct((n, dout), jnp.float32),
    )(x, W1a)

    epw8 = epw // 8
    ewr = -(-((NW - 1) * epw8 + nch * (CH // 8)) // 8) * 8
    eb = 2048
    w1b_big = jnp.kron(jnp.eye(8, dtype=jnp.float32), W1b)
    b1_t = jnp.tile(b1, 8).reshape(1, 8 * dout)
    ew = pl.pallas_call(
        _mm_bias_body,
        grid=(-(-ewr // eb),),
        in_specs=[pl.BlockSpec((eb, 8 * fe), lambda i: (i, 0)),
                  pl.BlockSpec((8 * fe, 8 * dout), lambda i: (0, 0)),
                  pl.BlockSpec((1, 8 * dout), lambda i: (0, 0))],
        out_specs=pl.BlockSpec((eb, 8 * dout), lambda i: (i, 0)),
        out_shape=jax.ShapeDtypeStruct((ewr, 8 * dout), jnp.float32),
    )(edge_attr.reshape(e // 8, 8 * fe), w1b_big, b1_t)

    mesh = plsc.VectorSubcoreMesh(core_axis_name="c", subcore_axis_name="s",
                                  num_cores=NC, num_subcores=NS)
    sc_fn = pl.kernel(
        functools.partial(_sc_scatter_body, nch, nps, npad, epw8),
        out_type=jax.ShapeDtypeStruct((NC * npad, ACCW), jnp.float32),
        mesh=mesh,
        compiler_params=pltpu.CompilerParams(use_tc_tiling_on_sc=False),
        scratch_types=[
            pltpu.VMEM((nch, CH), jnp.int32),
            pltpu.VMEM((nch, CH), jnp.int32),
            pltpu.VMEM((CH // 8, 8 * dout), jnp.float32),
            pltpu.VMEM((CH, dout), jnp.float32),
            pltpu.VMEM((CH, ACCW), jnp.float32),
            pltpu.VMEM((nps, ACCW), jnp.float32),
            pltpu.VMEM_SHARED((npad, ACCW), jnp.float32),
            pltpu.SemaphoreType.DMA,
        ],
    )
    parts = sc_fn(xw1, ew, row, col).reshape(NC, npad, ACCW)

    out = pl.pallas_call(
        functools.partial(_node_body, fx),
        grid=(n // rb,),
        in_specs=[pl.BlockSpec((rb, fx), lambda i: (i, 0)),
                  pl.BlockSpec((NC, rb, ACCW), lambda i: (0, i, 0)),
                  pl.BlockSpec((fx + dout, dout), lambda i: (0, 0)),
                  pl.BlockSpec((1, dout), lambda i: (0, 0))],
        out_specs=pl.BlockSpec((rb, dout), lambda i: (i, 0)),
        out_shape=jax.ShapeDtypeStruct((n, dout), jnp.float32),
    )(x, parts, W2, b2.reshape(1, dout))
    return out


# speedup vs baseline: 6.5107x; 1.1269x over previous
import functools

import jax
import jax.numpy as jnp
from jax import lax
from jax.experimental import pallas as pl
from jax.experimental.pallas import tpu as pltpu
from jax.experimental.pallas import tpu_sc as plsc

NC = 2
NS = 16
NW = NC * NS
L = 16
ACCW = 32
CH = 128


def _mm_body(x_ref, w_ref, o_ref):
    o_ref[...] = jnp.dot(x_ref[...], w_ref[...],
                         preferred_element_type=jnp.float32)


def _mm_bias_body(x_ref, w_ref, b_ref, o_ref):
    o_ref[...] = jnp.dot(x_ref[...], w_ref[...],
                         preferred_element_type=jnp.float32) + b_ref[...]


def _sc_scatter_body(NCH, NPS, npad, epw8,
                     xw1_hbm, ew_hbm, row_hbm, col_hbm, out_hbm,
                     row_buf, col_buf, ew_buf, gath_buf, val_buf, ones_buf,
                     vals_tmp, cnt_tmp, node_buf,
                     acc_sh, cnt_sh, sem, sem_c):
    c = lax.axis_index("c")
    s = lax.axis_index("s")
    wid = s * NC + c

    zvec = jnp.zeros((L,), jnp.float32)

    def zero_val(i, carry):
        vals_tmp[i] = zvec
        return carry

    lax.fori_loop(0, NPS, zero_val, 0)

    def zero_cnt(i, carry):
        cnt_tmp[pl.ds(i * L, L)] = zvec
        return carry

    lax.fori_loop(0, NPS // L, zero_cnt, 0)

    def one_row(i, carry):
        ones_buf[pl.ds(i * L, L)] = zvec + 1.0
        return carry

    lax.fori_loop(0, CH // L, one_row, 0)

    pltpu.sync_copy(vals_tmp, acc_sh.at[pl.ds(s * NPS, NPS)])
    pltpu.sync_copy(cnt_tmp, cnt_sh.at[pl.ds(s * NPS, NPS)])

    pltpu.sync_copy(row_hbm.at[wid], row_buf)
    pltpu.sync_copy(col_hbm.at[wid], col_buf)

    plsc.subcore_barrier()

    def chunk(k, carry):
        pltpu.sync_copy(
            ew_hbm.at[pl.ds(wid * epw8 + k * (CH // 8), CH // 8)], ew_buf)
        pltpu.async_copy(xw1_hbm.at[row_buf.at[k]], gath_buf, sem).wait()

        def pack8(j, carry2):
            for t in range(8):
                i = j * 8 + t
                v = gath_buf[i] + ew_buf[j, pl.ds(t * L, L)]
                val_buf[i] = jnp.maximum(v, 0.0)
            return carry2

        lax.fori_loop(0, CH // 8, pack8, 0)
        pltpu.async_copy(ones_buf, cnt_sh.at[col_buf.at[k]], sem_c, add=True)
        pltpu.sync_copy(val_buf, acc_sh.at[col_buf.at[k]], add=True)
        return carry

    lax.fori_loop(0, NCH, chunk, 0)

    def drain(k, carry):
        pltpu.make_async_copy(ones_buf, cnt_sh.at[col_buf.at[0]],
                              sem_c).wait()
        return carry

    lax.fori_loop(0, NCH, drain, 0)

    plsc.subcore_barrier()

    pltpu.sync_copy(acc_sh.at[pl.ds(s * NPS, NPS)], vals_tmp)
    pltpu.sync_copy(cnt_sh.at[pl.ds(s * NPS, NPS)], cnt_tmp)
    lane0 = lax.iota(jnp.int32, L) == 0

    def merge(g, carry):
        cvec = cnt_tmp[pl.ds(g * L, L)]
        for t in range(L):
            i = g * L + t
            node_buf[i, pl.ds(0, L)] = vals_tmp[i]
            node_buf[i, pl.ds(L, L)] = jnp.where(lane0, cvec[t], 0.0)
        return carry

    lax.fori_loop(0, NPS // L, merge, 0)
    pltpu.sync_copy(node_buf, out_hbm.at[pl.ds(c * npad + s * NPS, NPS)])


def _node_body(fx, x_ref, p_ref, w_ref, b_ref, o_ref):
    p0 = p_ref[0]
    p1 = p_ref[1]
    acc = p0[:, :L] + p1[:, :L]
    cnt = p0[:, L:L + 1] + p1[:, L:L + 1]
    mean = acc / jnp.maximum(cnt, 1.0)
    h = jnp.dot(x_ref[...], w_ref[:fx], preferred_element_type=jnp.float32)
    h = h + jnp.dot(mean, w_ref[fx:], preferred_element_type=jnp.float32)
    o_ref[...] = jnp.maximum(h + b_ref[...], 0.0)


def kernel(x, edge_index, edge_attr, u, batch, W1, b1, W2, b2):
    n, fx = x.shape
    e, fe = edge_attr.shape
    dout = W1.shape[1]
    assert dout == L and e % NW == 0 and fe == 16

    epw = e // NW
    nch = -(-epw // CH)
    epw_p = nch * CH
    npad = -(-n // (NS * L)) * (NS * L)
    nps = npad // NS
    assert npad > n

    ri = jnp.pad(edge_index[0].reshape(NW, epw), ((0, 0), (0, epw_p - epw)))
    row = ri.reshape(NW, nch, CH)
    ci = jnp.pad(edge_index[1].reshape(NW, epw), ((0, 0), (0, epw_p - epw)),
                 constant_values=n)
    col = ci.reshape(NW, nch, CH)

    W1a = W1[:fx]
    W1b = W1[fx:]

    rb = 1000
    xw1 = pl.pallas_call(
        _mm_body,
        grid=(n // rb,),
        in_specs=[pl.BlockSpec((rb, fx), lambda i: (i, 0)),
                  pl.BlockSpec((fx, dout), lambda i: (0, 0))],
        out_specs=pl.BlockSpec((rb, dout), lambda i: (i, 0)),
        out_shape=jax.ShapeDtypeStruct((n, dout), jnp.float32),
    )(x, W1a)

    epw8 = epw // 8
    ewr = -(-((NW - 1) * epw8 + nch * (CH // 8)) // 8) * 8
    eb = 2048
    w1b_big = jnp.kron(jnp.eye(8, dtype=jnp.float32), W1b)
    b1_t = jnp.tile(b1, 8).reshape(1, 8 * dout)
    ew = pl.pallas_call(
        _mm_bias_body,
        grid=(-(-ewr // eb),),
        in_specs=[pl.BlockSpec((eb, 8 * fe), lambda i: (i, 0)),
                  pl.BlockSpec((8 * fe, 8 * dout), lambda i: (0, 0)),
                  pl.BlockSpec((1, 8 * dout), lambda i: (0, 0))],
        out_specs=pl.BlockSpec((eb, 8 * dout), lambda i: (i, 0)),
        out_shape=jax.ShapeDtypeStruct((ewr, 8 * dout), jnp.float32),
    )(edge_attr.reshape(e // 8, 8 * fe), w1b_big, b1_t)

    mesh = plsc.VectorSubcoreMesh(core_axis_name="c", subcore_axis_name="s",
                                  num_cores=NC, num_subcores=NS)
    sc_fn = pl.kernel(
        functools.partial(_sc_scatter_body, nch, nps, npad, epw8),
        out_type=jax.ShapeDtypeStruct((NC * npad, ACCW), jnp.float32),
        mesh=mesh,
        compiler_params=pltpu.CompilerParams(use_tc_tiling_on_sc=False),
        scratch_types=[
            pltpu.VMEM((nch, CH), jnp.int32),
            pltpu.VMEM((nch, CH), jnp.int32),
            pltpu.VMEM((CH // 8, 8 * dout), jnp.float32),
            pltpu.VMEM((CH, dout), jnp.float32),
            pltpu.VMEM((CH, dout), jnp.float32),
            pltpu.VMEM((CH,), jnp.float32),
            pltpu.VMEM((nps, dout), jnp.float32),
            pltpu.VMEM((nps,), jnp.float32),
            pltpu.VMEM((nps, ACCW), jnp.float32),
            pltpu.VMEM_SHARED((npad, dout), jnp.float32),
            pltpu.VMEM_SHARED((npad,), jnp.float32),
            pltpu.SemaphoreType.DMA,
            pltpu.SemaphoreType.DMA,
        ],
    )
    parts = sc_fn(xw1, ew, row, col).reshape(NC, npad, ACCW)

    out = pl.pallas_call(
        functools.partial(_node_body, fx),
        grid=(n // rb,),
        in_specs=[pl.BlockSpec((rb, fx), lambda i: (i, 0)),
                  pl.BlockSpec((NC, rb, ACCW), lambda i: (0, i, 0)),
                  pl.BlockSpec((fx + dout, dout), lambda i: (0, 0)),
                  pl.BlockSpec((1, dout), lambda i: (0, 0))],
        out_specs=pl.BlockSpec((rb, dout), lambda i: (i, 0)),
        out_shape=jax.ShapeDtypeStruct((n, dout), jnp.float32),
    )(x, parts, W2, b2.reshape(1, dout))
    return out


# speedup vs baseline: 8.1090x; 1.2455x over previous
import functools

import jax
import jax.numpy as jnp
from jax import lax
from jax.experimental import pallas as pl
from jax.experimental.pallas import tpu as pltpu
from jax.experimental.pallas import tpu_sc as plsc

NC = 2
NS = 16
NW = NC * NS
L = 16
ACCW = 32
CH = 128


def _mm_body(x_ref, w_ref, o_ref):
    o_ref[...] = jnp.dot(x_ref[...], w_ref[...],
                         preferred_element_type=jnp.float32)


def _mm_bias_body(x_ref, w_ref, b_ref, o_ref):
    o_ref[...] = jnp.dot(x_ref[...], w_ref[...],
                         preferred_element_type=jnp.float32) + b_ref[...]


def _sc_scatter_body(NCH, NPS, npad, epw8,
                     xw1_hbm, ew_hbm, row_hbm, col_hbm, out_hbm,
                     row_buf, col_buf, ew_buf, ew_buf2, gath_buf, gath_buf2,
                     val_buf, val_buf2, ones_buf,
                     vals_tmp, cnt_tmp, node_buf,
                     acc_sh, cnt_sh,
                     sem_e0, sem_e1, sem_g0, sem_g1, sem_s0, sem_s1, sem_c):
    c = lax.axis_index("c")
    s = lax.axis_index("s")
    wid = s * NC + c

    zvec = jnp.zeros((L,), jnp.float32)

    def zero_val(i, carry):
        vals_tmp[i] = zvec
        return carry

    lax.fori_loop(0, NPS, zero_val, 0)

    def zero_cnt(i, carry):
        cnt_tmp[pl.ds(i * L, L)] = zvec
        return carry

    lax.fori_loop(0, NPS // L, zero_cnt, 0)

    def one_row(i, carry):
        ones_buf[pl.ds(i * L, L)] = zvec + 1.0
        return carry

    lax.fori_loop(0, CH // L, one_row, 0)

    pltpu.sync_copy(vals_tmp, acc_sh.at[pl.ds(s * NPS, NPS)])
    pltpu.sync_copy(cnt_tmp, cnt_sh.at[pl.ds(s * NPS, NPS)])

    pltpu.sync_copy(row_hbm.at[wid], row_buf)
    pltpu.sync_copy(col_hbm.at[wid], col_buf)

    plsc.subcore_barrier()

    ew_bufs = (ew_buf, ew_buf2)
    gath_bufs = (gath_buf, gath_buf2)
    val_bufs = (val_buf, val_buf2)
    sem_es = (sem_e0, sem_e1)
    sem_gs = (sem_g0, sem_g1)
    sem_ss = (sem_s0, sem_s1)

    def issue_loads(k, b):
        pltpu.async_copy(
            ew_hbm.at[pl.ds(wid * epw8 + k * (CH // 8), CH // 8)],
            ew_bufs[b], sem_es[b])
        pltpu.async_copy(xw1_hbm.at[row_buf.at[k]], gath_bufs[b], sem_gs[b])

    issue_loads(0, 0)

    def chunk_pair(k2, carry):
        for b in range(2):
            k = 2 * k2 + b
            nb = 1 - b
            if b == 0:
                issue_loads(k + 1, nb)
            else:
                @pl.when(k2 < NCH // 2 - 1)
                def _():
                    issue_loads(k + 1, nb)
            pltpu.make_async_copy(
                ew_hbm.at[pl.ds(wid * epw8 + k * (CH // 8), CH // 8)],
                ew_bufs[b], sem_es[b]).wait()
            pltpu.make_async_copy(xw1_hbm.at[row_buf.at[k]], gath_bufs[b],
                                  sem_gs[b]).wait()

            @pl.when(k2 >= 1)
            def _():
                pltpu.make_async_copy(val_bufs[b],
                                      acc_sh.at[col_buf.at[k]],
                                      sem_ss[b]).wait()

            def pack8(j, carry2, _b=b):
                for t in range(8):
                    i = j * 8 + t
                    v = gath_bufs[_b][i] + ew_bufs[_b][j, pl.ds(t * L, L)]
                    val_bufs[_b][i] = jnp.maximum(v, 0.0)
                return carry2

            lax.fori_loop(0, CH // 8, pack8, 0)
            pltpu.async_copy(ones_buf, cnt_sh.at[col_buf.at[k]], sem_c,
                             add=True)
            pltpu.async_copy(val_bufs[b], acc_sh.at[col_buf.at[k]],
                             sem_ss[b], add=True)
        return carry

    lax.fori_loop(0, NCH // 2, chunk_pair, 0)

    for b in range(2):
        pltpu.make_async_copy(val_bufs[b], acc_sh.at[col_buf.at[0]],
                              sem_ss[b]).wait()

    def drain(k, carry):
        pltpu.make_async_copy(ones_buf, cnt_sh.at[col_buf.at[0]],
                              sem_c).wait()
        return carry

    lax.fori_loop(0, NCH, drain, 0)

    plsc.subcore_barrier()

    pltpu.sync_copy(acc_sh.at[pl.ds(s * NPS, NPS)], vals_tmp)
    pltpu.sync_copy(cnt_sh.at[pl.ds(s * NPS, NPS)], cnt_tmp)
    lane0 = lax.iota(jnp.int32, L) == 0

    def merge(g, carry):
        cvec = cnt_tmp[pl.ds(g * L, L)]
        for t in range(L):
            i = g * L + t
            node_buf[i, pl.ds(0, L)] = vals_tmp[i]
            node_buf[i, pl.ds(L, L)] = jnp.where(lane0, cvec[t], 0.0)
        return carry

    lax.fori_loop(0, NPS // L, merge, 0)
    pltpu.sync_copy(node_buf, out_hbm.at[pl.ds(c * npad + s * NPS, NPS)])


def _node_body(fx, x_ref, p_ref, w_ref, b_ref, o_ref):
    p0 = p_ref[0]
    p1 = p_ref[1]
    acc = p0[:, :L] + p1[:, :L]
    cnt = p0[:, L:L + 1] + p1[:, L:L + 1]
    mean = acc / jnp.maximum(cnt, 1.0)
    h = jnp.dot(x_ref[...], w_ref[:fx], preferred_element_type=jnp.float32)
    h = h + jnp.dot(mean, w_ref[fx:], preferred_element_type=jnp.float32)
    o_ref[...] = jnp.maximum(h + b_ref[...], 0.0)


def kernel(x, edge_index, edge_attr, u, batch, W1, b1, W2, b2):
    n, fx = x.shape
    e, fe = edge_attr.shape
    dout = W1.shape[1]
    assert dout == L and e % NW == 0 and fe == 16

    epw = e // NW
    nch = -(-epw // CH)
    nch += nch & 1
    epw_p = nch * CH
    npad = -(-n // (NS * L)) * (NS * L)
    nps = npad // NS
    assert npad > n

    ri = jnp.pad(edge_index[0].reshape(NW, epw), ((0, 0), (0, epw_p - epw)))
    row = ri.reshape(NW, nch, CH)
    ci = jnp.pad(edge_index[1].reshape(NW, epw), ((0, 0), (0, epw_p - epw)),
                 constant_values=n)
    col = ci.reshape(NW, nch, CH)

    W1a = W1[:fx]
    W1b = W1[fx:]

    rb = 1000
    xw1 = pl.pallas_call(
        _mm_body,
        grid=(n // rb,),
        in_specs=[pl.BlockSpec((rb, fx), lambda i: (i, 0)),
                  pl.BlockSpec((fx, dout), lambda i: (0, 0))],
        out_specs=pl.BlockSpec((rb, dout), lambda i: (i, 0)),
        out_shape=jax.ShapeDtypeStruct((n, dout), jnp.float32),
    )(x, W1a)

    epw8 = epw // 8
    ewr = -(-((NW - 1) * epw8 + nch * (CH // 8)) // 8) * 8
    eb = 2048
    w1b_big = jnp.kron(jnp.eye(8, dtype=jnp.float32), W1b)
    b1_t = jnp.tile(b1, 8).reshape(1, 8 * dout)
    ew = pl.pallas_call(
        _mm_bias_body,
        grid=(-(-ewr // eb),),
        in_specs=[pl.BlockSpec((eb, 8 * fe), lambda i: (i, 0)),
                  pl.BlockSpec((8 * fe, 8 * dout), lambda i: (0, 0)),
                  pl.BlockSpec((1, 8 * dout), lambda i: (0, 0))],
        out_specs=pl.BlockSpec((eb, 8 * dout), lambda i: (i, 0)),
        out_shape=jax.ShapeDtypeStruct((ewr, 8 * dout), jnp.float32),
    )(edge_attr.reshape(e // 8, 8 * fe), w1b_big, b1_t)

    mesh = plsc.VectorSubcoreMesh(core_axis_name="c", subcore_axis_name="s",
                                  num_cores=NC, num_subcores=NS)
    sc_fn = pl.kernel(
        functools.partial(_sc_scatter_body, nch, nps, npad, epw8),
        out_type=jax.ShapeDtypeStruct((NC * npad, ACCW), jnp.float32),
        mesh=mesh,
        compiler_params=pltpu.CompilerParams(use_tc_tiling_on_sc=False),
        scratch_types=[
            pltpu.VMEM((nch, CH), jnp.int32),
            pltpu.VMEM((nch, CH), jnp.int32),
            pltpu.VMEM((CH // 8, 8 * dout), jnp.float32),
            pltpu.VMEM((CH // 8, 8 * dout), jnp.float32),
            pltpu.VMEM((CH, dout), jnp.float32),
            pltpu.VMEM((CH, dout), jnp.float32),
            pltpu.VMEM((CH, dout), jnp.float32),
            pltpu.VMEM((CH, dout), jnp.float32),
            pltpu.VMEM((CH,), jnp.float32),
            pltpu.VMEM((nps, dout), jnp.float32),
            pltpu.VMEM((nps,), jnp.float32),
            pltpu.VMEM((nps, ACCW), jnp.float32),
            pltpu.VMEM_SHARED((npad, dout), jnp.float32),
            pltpu.VMEM_SHARED((npad,), jnp.float32),
        ] + [pltpu.SemaphoreType.DMA] * 7,
    )
    parts = sc_fn(xw1, ew, row, col).reshape(NC, npad, ACCW)

    out = pl.pallas_call(
        functools.partial(_node_body, fx),
        grid=(n // rb,),
        in_specs=[pl.BlockSpec((rb, fx), lambda i: (i, 0)),
                  pl.BlockSpec((NC, rb, ACCW), lambda i: (0, i, 0)),
                  pl.BlockSpec((fx + dout, dout), lambda i: (0, 0)),
                  pl.BlockSpec((1, dout), lambda i: (0, 0))],
        out_specs=pl.BlockSpec((rb, dout), lambda i: (i, 0)),
        out_shape=jax.ShapeDtypeStruct((n, dout), jnp.float32),
    )(x, parts, W2, b2.reshape(1, dout))
    return out


# speedup vs baseline: 8.3777x; 1.0331x over previous
import functools

import jax
import jax.numpy as jnp
from jax import lax
from jax.experimental import pallas as pl
from jax.experimental.pallas import tpu as pltpu
from jax.experimental.pallas import tpu_sc as plsc

NC = 2
NS = 16
NW = NC * NS
L = 16
CH = 128


def _stage_a_body(neb, ea_ref, wb_ref, b_ref, x_ref, wa_ref,
                  ew_ref, xw1_ref):
    i = pl.program_id(0)

    @pl.when(i < neb)
    def _():
        ew_ref[...] = jnp.dot(ea_ref[...], wb_ref[...],
                              preferred_element_type=jnp.float32) + b_ref[...]

    @pl.when(i == neb)
    def _():
        xw1_ref[...] = jnp.dot(x_ref[...], wa_ref[...],
                               preferred_element_type=jnp.float32)


def _sc_scatter_body(NCH, NPS, npad, epw8,
                     xw1_hbm, ew_hbm, row_hbm, col_hbm,
                     outv_hbm,
                     row_buf, col_buf, ew_buf, ew_buf2, gath_buf, gath_buf2,
                     val_buf, val_buf2, ones_buf,
                     vals_tmp, cnt_tmp, node_buf,
                     acc_sh, cnt_sh,
                     sem_e0, sem_e1, sem_g0, sem_g1, sem_s0, sem_s1, sem_c):
    c = lax.axis_index("c")
    s = lax.axis_index("s")
    wid = s * NC + c

    zvec = jnp.zeros((L,), jnp.float32)

    def zero_val(i, carry):
        vals_tmp[i] = zvec
        return carry

    lax.fori_loop(0, NPS, zero_val, 0)

    def zero_cnt(i, carry):
        cnt_tmp[pl.ds(i * L, L)] = zvec
        return carry

    lax.fori_loop(0, NPS // L, zero_cnt, 0)

    def one_row(i, carry):
        ones_buf[pl.ds(i * L, L)] = zvec + 1.0
        return carry

    lax.fori_loop(0, CH // L, one_row, 0)

    pltpu.sync_copy(vals_tmp, acc_sh.at[pl.ds(s * NPS, NPS)])
    pltpu.sync_copy(cnt_tmp, cnt_sh.at[pl.ds(s * NPS, NPS)])

    pltpu.sync_copy(row_hbm.at[wid], row_buf)
    pltpu.sync_copy(col_hbm.at[wid], col_buf)

    plsc.subcore_barrier()

    ew_bufs = (ew_buf, ew_buf2)
    gath_bufs = (gath_buf, gath_buf2)
    val_bufs = (val_buf, val_buf2)
    sem_es = (sem_e0, sem_e1)
    sem_gs = (sem_g0, sem_g1)
    sem_ss = (sem_s0, sem_s1)

    def issue_loads(k, b):
        pltpu.async_copy(
            ew_hbm.at[pl.ds(wid * epw8 + k * (CH // 8), CH // 8)],
            ew_bufs[b], sem_es[b])
        pltpu.async_copy(xw1_hbm.at[row_buf.at[k]], gath_bufs[b], sem_gs[b])

    issue_loads(0, 0)

    def chunk_pair(k2, carry):
        for b in range(2):
            k = 2 * k2 + b
            nb = 1 - b
            if b == 0:
                issue_loads(k + 1, nb)
            else:
                @pl.when(k2 < NCH // 2 - 1)
                def _():
                    issue_loads(k + 1, nb)
            pltpu.make_async_copy(
                ew_hbm.at[pl.ds(wid * epw8 + k * (CH // 8), CH // 8)],
                ew_bufs[b], sem_es[b]).wait()
            pltpu.make_async_copy(xw1_hbm.at[row_buf.at[k]], gath_bufs[b],
                                  sem_gs[b]).wait()

            @pl.when(k2 >= 1)
            def _():
                pltpu.make_async_copy(val_bufs[b],
                                      acc_sh.at[col_buf.at[k]],
                                      sem_ss[b]).wait()

            def pack8(j, carry2, _b=b):
                for t in range(8):
                    i = j * 8 + t
                    v = gath_bufs[_b][i] + ew_bufs[_b][j, pl.ds(t * L, L)]
                    val_bufs[_b][i] = jnp.maximum(v, 0.0)
                return carry2

            lax.fori_loop(0, CH // 8, pack8, 0)
            pltpu.async_copy(ones_buf, cnt_sh.at[col_buf.at[k]], sem_c,
                             add=True)
            pltpu.async_copy(val_bufs[b], acc_sh.at[col_buf.at[k]],
                             sem_ss[b], add=True)
        return carry

    lax.fori_loop(0, NCH // 2, chunk_pair, 0)

    for b in range(2):
        pltpu.make_async_copy(val_bufs[b], acc_sh.at[col_buf.at[0]],
                              sem_ss[b]).wait()

    def drain(k, carry):
        pltpu.make_async_copy(ones_buf, cnt_sh.at[col_buf.at[0]],
                              sem_c).wait()
        return carry

    lax.fori_loop(0, NCH, drain, 0)

    plsc.subcore_barrier()

    pltpu.sync_copy(acc_sh.at[pl.ds(s * NPS, NPS)], vals_tmp)
    pltpu.sync_copy(cnt_sh.at[pl.ds(s * NPS, NPS)], cnt_tmp)
    lane0 = lax.iota(jnp.int32, L) == 0

    def merge(g, carry):
        cvec = cnt_tmp[pl.ds(g * L, L)]
        for t in range(L):
            i = g * L + t
            node_buf[i, pl.ds(0, L)] = vals_tmp[i]
            node_buf[i, pl.ds(L, L)] = jnp.where(lane0, cvec[t], 0.0)
        return carry

    lax.fori_loop(0, NPS // L, merge, 0)
    pltpu.sync_copy(node_buf, outv_hbm.at[pl.ds(c * npad + s * NPS, NPS)])


def _node_body(x_ref, p_ref, w_ref, b_ref, o_ref):
    fx = x_ref.shape[1]
    p0 = p_ref[0]
    p1 = p_ref[1]
    acc = p0[:, :L] + p1[:, :L]
    cnt = p0[:, L:L + 1] + p1[:, L:L + 1]
    mean = acc / jnp.maximum(cnt, 1.0)
    h = jnp.dot(x_ref[...], w_ref[:fx], preferred_element_type=jnp.float32)
    h = h + jnp.dot(mean, w_ref[fx:], preferred_element_type=jnp.float32)
    o_ref[...] = jnp.maximum(h + b_ref[...], 0.0)


def kernel(x, edge_index, edge_attr, u, batch, W1, b1, W2, b2):
    n, fx = x.shape
    e, fe = edge_attr.shape
    dout = W1.shape[1]
    assert dout == L and e % NW == 0 and fe == 16 and n % 8 == 0

    epw = e // NW
    nch = -(-epw // CH)
    nch += nch & 1
    epw_p = nch * CH
    npad = -(-n // (NS * L)) * (NS * L)
    nps = npad // NS
    assert npad > n

    ri = jnp.pad(edge_index[0].reshape(NW, epw), ((0, 0), (0, epw_p - epw)))
    row = ri.reshape(NW, nch, CH)
    ci = jnp.pad(edge_index[1].reshape(NW, epw), ((0, 0), (0, epw_p - epw)),
                 constant_values=n)
    col = ci.reshape(NW, nch, CH)

    eye8 = jnp.eye(8, dtype=jnp.float32)
    rb = 1024

    epw8 = epw // 8
    ewr = -(-((NW - 1) * epw8 + nch * (CH // 8)) // 8) * 8
    eb = 2048
    neb = -(-ewr // eb)
    w1b_big = jnp.kron(eye8, W1[fx:])
    b1_t = jnp.tile(b1, 8).reshape(1, 8 * dout)
    ew, xw1 = pl.pallas_call(
        functools.partial(_stage_a_body, neb),
        grid=(neb + 1,),
        in_specs=[
            pl.BlockSpec((eb, 8 * fe), lambda i: (jnp.minimum(i, neb - 1), 0)),
            pl.BlockSpec((8 * fe, 8 * dout), lambda i: (0, 0)),
            pl.BlockSpec((1, 8 * dout), lambda i: (0, 0)),
            pl.BlockSpec((n, fx), lambda i: (0, 0)),
            pl.BlockSpec((fx, dout), lambda i: (0, 0)),
        ],
        out_specs=[
            pl.BlockSpec((eb, 8 * dout), lambda i: (jnp.minimum(i, neb - 1), 0)),
            pl.BlockSpec((n, dout), lambda i: (0, 0)),
        ],
        out_shape=[
            jax.ShapeDtypeStruct((ewr, 8 * dout), jnp.float32),
            jax.ShapeDtypeStruct((n, dout), jnp.float32),
        ],
    )(edge_attr.reshape(e // 8, 8 * fe), w1b_big, b1_t, x, W1[:fx])

    mesh = plsc.VectorSubcoreMesh(core_axis_name="c", subcore_axis_name="s",
                                  num_cores=NC, num_subcores=NS)
    sc_fn = pl.kernel(
        functools.partial(_sc_scatter_body, nch, nps, npad, epw8),
        out_type=jax.ShapeDtypeStruct((NC * npad, 2 * dout), jnp.float32),
        mesh=mesh,
        compiler_params=pltpu.CompilerParams(use_tc_tiling_on_sc=False),
        scratch_types=[
            pltpu.VMEM((nch, CH), jnp.int32),
            pltpu.VMEM((nch, CH), jnp.int32),
            pltpu.VMEM((CH // 8, 8 * dout), jnp.float32),
            pltpu.VMEM((CH // 8, 8 * dout), jnp.float32),
            pltpu.VMEM((CH, dout), jnp.float32),
            pltpu.VMEM((CH, dout), jnp.float32),
            pltpu.VMEM((CH, dout), jnp.float32),
            pltpu.VMEM((CH, dout), jnp.float32),
            pltpu.VMEM((CH,), jnp.float32),
            pltpu.VMEM((nps, dout), jnp.float32),
            pltpu.VMEM((nps,), jnp.float32),
            pltpu.VMEM((nps, 2 * dout), jnp.float32),
            pltpu.VMEM_SHARED((npad, dout), jnp.float32),
            pltpu.VMEM_SHARED((npad,), jnp.float32),
        ] + [pltpu.SemaphoreType.DMA] * 7,
    )
    parts = sc_fn(xw1, ew, row, col).reshape(NC, npad, 2 * dout)

    out = pl.pallas_call(
        _node_body,
        grid=(-(-n // rb),),
        in_specs=[pl.BlockSpec((rb, fx), lambda i: (i, 0)),
                  pl.BlockSpec((NC, rb, 2 * dout), lambda i: (0, i, 0)),
                  pl.BlockSpec((fx + dout, dout), lambda i: (0, 0)),
                  pl.BlockSpec((1, dout), lambda i: (0, 0))],
        out_specs=pl.BlockSpec((rb, dout), lambda i: (i, 0)),
        out_shape=jax.ShapeDtypeStruct((n, dout), jnp.float32),
    )(x, parts, W2, b2.reshape(1, dout))
    return out
